# baseline (speedup 1.0000x reference)
import functools

import jax
import jax.numpy as jnp
from jax import lax
from jax.experimental import pallas as pl
from jax.experimental.pallas import tpu as pltpu
from jax.experimental.pallas import tpu_sc as plsc

NC = 2
NS = 16
L = 16
NW = NC * NS

_f32 = jnp.float32
_i32 = jnp.int32


def _sc_mesh():
    return plsc.VectorSubcoreMesh(core_axis_name="c", subcore_axis_name="s")


_SC_PARAMS = pltpu.CompilerParams(needs_layout_passes=False,
                                  use_tc_tiling_on_sc=False,
                                  skip_device_barrier=True)


@functools.lru_cache(maxsize=None)
def _make_w_kernel(Npad, ETp, C, CPT):
    per_tile = CPT * C

    assert CPT % 2 == 0

    def body(s1_hbm, d1_hbm, src_hbm, dst_hbm, w_hbm, s1_v, d1_v, src0,
             src1, dst0, dst1, w0, w1, isem, wsem0, wsem1):
        srcb = (src0, src1)
        dstb = (dst0, dst1)
        wb = (w0, w1)
        wsem = (wsem0, wsem1)
        cid = lax.axis_index("c")
        sid = lax.axis_index("s")
        base_r = (cid * NS + sid) * CPT
        pltpu.sync_copy(s1_hbm, s1_v)
        pltpu.sync_copy(d1_hbm, d1_v)

        def load_idx(j, b, sync):
            r = base_r + j
            cps = [pltpu.make_async_copy(src_hbm.at[r], srcb[b], isem),
                   pltpu.make_async_copy(dst_hbm.at[r], dstb[b], isem)]
            for cp in cps:
                cp.start()
            if sync:
                for cp in cps:
                    cp.wait()

        def wait_idx(b):
            pltpu.make_async_copy(src_hbm.at[0], srcb[b], isem).wait()
            pltpu.make_async_copy(dst_hbm.at[0], dstb[b], isem).wait()

        load_idx(0, 0, True)
        load_idx(1, 1, False)

        def pair(k, carry):
            for b in range(2):
                j = 2 * k + b

                @pl.when(k > 0)
                def _():
                    pltpu.make_async_copy(
                        wb[b], w_hbm.at[pl.ds(0, C)], wsem[b]).wait()

                for q in range(C // L):
                    si = srcb[b][pl.ds(q * L, L)]
                    di = dstb[b][pl.ds(q * L, L)]
                    sv = plsc.load_gather(s1_v, [si])
                    dv = plsc.load_gather(d1_v, [di])
                    e = 1.0 / (1.0 + jnp.exp(-(sv + dv)))
                    wb[b][pl.ds(q * L, L)] = jnp.exp(e)
                pltpu.async_copy(
                    wb[b], w_hbm.at[pl.ds((base_r + j) * C, C)], wsem[b])

                @pl.when(2 * k + 1 + b < CPT)
                def _():
                    wait_idx(1 - b)

                @pl.when(2 * k + 2 + b < CPT)
                def _():
                    load_idx(j + 2, b, False)
            return carry

        lax.fori_loop(0, CPT // 2, pair, 0)
        for b in range(2):
            pltpu.make_async_copy(wb[b], w_hbm.at[pl.ds(0, C)],
                                  wsem[b]).wait()

    return pl.kernel(
        body,
        out_type=jax.ShapeDtypeStruct((ETp,), _f32),
        mesh=_sc_mesh(),
        compiler_params=_SC_PARAMS,
        scratch_types=[
            pltpu.VMEM((Npad,), _f32),
            pltpu.VMEM((Npad,), _f32),
            pltpu.VMEM((C,), _i32),
            pltpu.VMEM((C,), _i32),
            pltpu.VMEM((C,), _i32),
            pltpu.VMEM((C,), _i32),
            pltpu.VMEM((C,), _f32),
            pltpu.VMEM((C,), _f32),
            pltpu.SemaphoreType.DMA,
            pltpu.SemaphoreType.DMA,
            pltpu.SemaphoreType.DMA,
        ],
    )


@functools.lru_cache(maxsize=None)
def _make_spmm_kernel(Npad, ETp, C, CPT, K, with_w, want_z=True):
    assert C == 128 and CPT % 6 == 0
    RPT = Npad // NS
    SB = 128
    NB = RPT // SB

    def body(h_hbm, src_hbm, dst_hbm, *rest):
        if with_w:
            w_hbm, rest = rest[0], rest[1:]
        else:
            w_hbm = None
        if want_z:
            acc_out, z_out, rest = rest[0], rest[1], rest[2:]
            z_sh, zst_v, rest = rest[0], rest[1], rest[2:]
        else:
            acc_out, rest = rest[0], rest[1:]
            z_out = z_sh = zst_v = None
        (acc_sh, rows0, rows1,
         si0, si1, si2, di0, di1, di2, w0, w1, w2,
         gsem0, gsem1, ssem0, ssem1, isem) = rest
        h_src = h_hbm
        stage_v = rows0.at[pl.ds(0, SB)]
        rows = (rows0, rows1)
        si = (si0, si1, si2)
        di = (di0, di1, di2)
        wv = (w0, w1, w2)
        gsem = (gsem0, gsem1)
        ssem = (ssem0, ssem1)
        cid = lax.axis_index("c")
        sid = lax.axis_index("s")
        base_r = (cid * NS + sid) * CPT
        myrow = sid * RPT
        zero = jnp.zeros((L,), _f32)

        def zr(r, carry):
            for q in range(K // L):
                rows0[r, pl.ds(q * L, L)] = zero
            return carry

        lax.fori_loop(0, SB, zr, 0)
        for b in range(NB):
            pltpu.sync_copy(stage_v, acc_sh.at[pl.ds(myrow + b * SB, SB)])

        if want_z:
            def zz(r, carry):
                zst_v[pl.ds(r * L, L)] = zero
                return carry

            lax.fori_loop(0, RPT // L, zz, 0)
            pltpu.sync_copy(zst_v, z_sh.at[pl.ds(myrow, RPT)])

        if not with_w and want_z:
            one = jnp.full((L,), 1.0, _f32)

            def oo(r, carry):
                w0[pl.ds(r * L, L)] = one
                w1[pl.ds(r * L, L)] = one
                w2[pl.ds(r * L, L)] = one
                return carry

            lax.fori_loop(0, C // L, oo, 0)

        plsc.subcore_barrier()

        def load_idx(j, s3, sync):
            r = base_r + j
            cps = [pltpu.make_async_copy(src_hbm.at[r], si[s3], isem),
                   pltpu.make_async_copy(dst_hbm.at[r], di[s3], isem)]
            if with_w:
                cps.append(pltpu.make_async_copy(
                    w_hbm.at[pl.ds(r * C, C)], wv[s3], isem))
            for cp in cps:
                cp.start()
            if sync:
                for cp in cps:
                    cp.wait()

        def wait_idx(s3):
            pltpu.make_async_copy(src_hbm.at[0], si[s3], isem).wait()
            pltpu.make_async_copy(dst_hbm.at[0], di[s3], isem).wait()
            if with_w:
                pltpu.make_async_copy(w_hbm.at[pl.ds(0, C)], wv[s3],
                                      isem).wait()

        def start_gather(b2, s3):
            pltpu.async_copy(h_src.at[si[s3]], rows[b2], gsem[b2])

        def wait_gather(b2, s3):
            pltpu.make_async_copy(h_src.at[si[s3]], rows[b2],
                                  gsem[b2]).wait()

        def start_scatter(b2, s3):
            pltpu.async_copy(rows[b2], acc_sh.at[di[s3]], ssem[b2],
                             add=True)
            if want_z:
                pltpu.async_copy(wv[s3], z_sh.at[di[s3]], ssem[b2],
                                 add=True)

        def wait_scatter(b2, s3):
            pltpu.make_async_copy(rows[b2], acc_sh.at[di[s3]],
                                  ssem[b2]).wait()
            if want_z:
                pltpu.make_async_copy(wv[s3], z_sh.at[di[s3]],
                                      ssem[b2]).wait()

        load_idx(0, 0, True)
        start_gather(0, 0)
        load_idx(1, 1, False)

        def six(k, carry):
            for u in range(6):
                j = 6 * k + u
                b2 = u % 2
                s3 = u % 3
                wait_gather(b2, s3)
                if u == 0:
                    @pl.when(k > 0)
                    def _():
                        wait_scatter(1 - b2, (u + 2) % 3)
                else:
                    wait_scatter(1 - b2, (u + 2) % 3)

                @pl.when(j + 1 < CPT)
                def _():
                    wait_idx((u + 1) % 3)
                    start_gather(1 - b2, (u + 1) % 3)

                @pl.when(j + 2 < CPT)
                def _():
                    load_idx(j + 2, (u + 2) % 3, False)

                if with_w:
                    zvec = jnp.zeros((L,), _i32)

                    def sc_row(r, c2):
                        wspl = plsc.load_gather(wv[s3], [zvec + r])
                        for q in range(K // L):
                            rows[b2][r, pl.ds(q * L, L)] = (
                                rows[b2][r, pl.ds(q * L, L)] * wspl)
                        return c2

                    lax.fori_loop(0, C, sc_row, 0, unroll=4)
                start_scatter(b2, s3)
            return carry

        lax.fori_loop(0, CPT // 6, six, 0)
        wait_scatter((CPT - 1) % 2, (CPT - 1) % 3)
        plsc.subcore_barrier()

        pltpu.sync_copy(acc_sh.at[pl.ds(myrow, RPT)],
                        acc_out.at[cid, pl.ds(myrow, RPT)])
        if want_z:
            pltpu.sync_copy(z_sh.at[pl.ds(myrow, RPT)],
                            z_out.at[cid, pl.ds(myrow, RPT)])

    scratch = ([pltpu.VMEM_SHARED((Npad,), _f32),
                pltpu.VMEM((RPT,), _f32)] if want_z else []) + [
        pltpu.VMEM_SHARED((Npad, K), _f32),
        pltpu.VMEM((C, K), _f32),
        pltpu.VMEM((C, K), _f32),
        pltpu.VMEM((C,), _i32),
        pltpu.VMEM((C,), _i32),
        pltpu.VMEM((C,), _i32),
        pltpu.VMEM((C,), _i32),
        pltpu.VMEM((C,), _i32),
        pltpu.VMEM((C,), _i32),
        pltpu.VMEM((C,), _f32),
        pltpu.VMEM((C,), _f32),
        pltpu.VMEM((C,), _f32),
        pltpu.SemaphoreType.DMA,
        pltpu.SemaphoreType.DMA,
        pltpu.SemaphoreType.DMA,
        pltpu.SemaphoreType.DMA,
        pltpu.SemaphoreType.DMA,
    ]
    out_type = jax.ShapeDtypeStruct((NC, Npad, K), _f32)
    if want_z:
        out_type = (out_type, jax.ShapeDtypeStruct((NC, Npad), _f32))
    return pl.kernel(
        body,
        out_type=out_type,
        mesh=_sc_mesh(),
        compiler_params=_SC_PARAMS,
        scratch_types=scratch,
    )


def _tc1(x_ref, ws_ref, wd_ref, as_ref, ad_ref, hs_ref, s1_ref, d1_ref):
    x = x_ref[...]
    hs = jnp.dot(x, ws_ref[...], preferred_element_type=_f32)
    hs_ref[...] = hs
    s1_ref[...] = jnp.dot(hs, as_ref[...], preferred_element_type=_f32)
    vd = jnp.dot(wd_ref[...], ad_ref[...], preferred_element_type=_f32)
    d1_ref[...] = jnp.dot(x, vd, preferred_element_type=_f32)


def _tc2(P_ref, z_ref, b1_ref, w2_ref, out_ref):
    Ps = P_ref[0] + P_ref[1]
    z = z_ref[0] + z_ref[1]
    h1 = Ps * (1.0 / z) + b1_ref[...]
    h1 = jnp.where(h1 > 0, h1, jnp.exp(jnp.minimum(h1, 0.0)) - 1.0)
    out_ref[...] = jnp.dot(h1, w2_ref[...], preferred_element_type=_f32)


def _tc3(Q_ref, zd_ref, b2_ref, w2t_ref, out_ref):
    agg = Q_ref[0] + Q_ref[1]
    deg = zd_ref[0] + zd_ref[1]
    h2 = agg * (1.0 / deg) + b2_ref[...]
    nrm = jnp.sqrt(jnp.sum(h2 * h2, axis=1, keepdims=True))
    h2 = h2 / jnp.maximum(nrm, 1e-12)
    out_ref[...] = jnp.dot(h2, w2t_ref[...], preferred_element_type=_f32)


def _tc4(R_ref, z_ref, b3_ref, w1t_ref, out_ref):
    Rs = R_ref[0] + R_ref[1]
    z = z_ref[0] + z_ref[1]
    h3 = Rs * (1.0 / z) + b3_ref[...]
    h3 = jnp.where(h3 > 0, h3, jnp.exp(jnp.minimum(h3, 0.0)) - 1.0)
    out_ref[...] = jnp.dot(h3, w1t_ref[...], preferred_element_type=_f32)


def _make_tc5(N, D):
    def _tc5(S_ref, zd_ref, b4_ref, x_ref, loss_ref):
        agg = S_ref[0, :N, :] + S_ref[1, :N, :]
        deg = zd_ref[0, :N, :] + zd_ref[1, :N, :]
        h4 = agg * (1.0 / deg) + b4_ref[...]
        r = x_ref[...] - h4
        loss_ref[0, 0] = jnp.sum(r * r) / (N * D)

    return _tc5


def kernel(features, edge_index, W1_src, W1_dst, a1_src, a1_dst, b1,
           W2_src, W2_dst, b2, b3, b4):
    N, D = features.shape
    F1 = W1_src.shape[1]
    F2 = W2_src.shape[1]
    E = edge_index.shape[1]
    Npad = -(-N // 2048) * 2048
    C = 128
    Etot = E + N
    CPT = -(-Etot // (NW * C))
    CPT = -(-CPT // 6) * 6
    ETp = NW * C * CPT
    NCH = ETp // 128
    npadE = ETp - Etot

    xp = jnp.zeros((Npad, D), _f32).at[:N].set(features)
    loop = jnp.arange(N, dtype=_i32)
    pad_src = jnp.arange(npadE, dtype=_i32) % N
    pad_dst = N + jnp.arange(npadE, dtype=_i32) % (Npad - N)
    src = jnp.concatenate([edge_index[0].astype(_i32), loop,
                           pad_src]).reshape(NCH, 128)
    dst = jnp.concatenate([edge_index[1].astype(_i32), loop,
                           pad_dst]).reshape(NCH, 128)
    a_s = a1_src.reshape(F1, 1)
    a_d = a1_dst.reshape(F1, 1)
    b1r = b1.reshape(1, F1)
    b2r = b2.reshape(1, F2)
    b3r = b3.reshape(1, F1)
    b4r = b4.reshape(1, D)
    W2T = W2_src.T
    W1T = W1_src.T

    hs1, s1, d1 = pl.pallas_call(
        _tc1,
        out_shape=(jax.ShapeDtypeStruct((Npad, F1), _f32),
                   jax.ShapeDtypeStruct((Npad, 1), _f32),
                   jax.ShapeDtypeStruct((Npad, 1), _f32)),
    )(xp, W1_src, W1_dst, a_s, a_d)

    w = _make_w_kernel(Npad, ETp, 128, ETp // (NW * 128))(
        s1.reshape(Npad), d1.reshape(Npad), src, dst)

    P, z = _make_spmm_kernel(Npad, ETp, C, CPT, F1, True)(hs1, src, dst, w)

    h2pre = pl.pallas_call(
        _tc2, out_shape=jax.ShapeDtypeStruct((Npad, F2), _f32),
    )(P, z.reshape(NC, Npad, 1), b1r, W2_src)

    Q, deg = _make_spmm_kernel(Npad, ETp, C, CPT, F2, False)(h2pre, src, dst)

    h3pre = pl.pallas_call(
        _tc3, out_shape=jax.ShapeDtypeStruct((Npad, F1), _f32),
    )(Q, deg.reshape(NC, Npad, 1), b2r, W2T)

    R = _make_spmm_kernel(Npad, ETp, C, CPT, F1, True,
                          want_z=False)(h3pre, src, dst, w)

    h4pre = pl.pallas_call(
        _tc4, out_shape=jax.ShapeDtypeStruct((Npad, D), _f32),
    )(R, z.reshape(NC, Npad, 1), b3r, W1T)

    S = _make_spmm_kernel(Npad, ETp, C, CPT, D, False,
                          want_z=False)(h4pre, src, dst)

    loss = pl.pallas_call(
        _make_tc5(N, D),
        out_shape=jax.ShapeDtypeStruct((1, 1), _f32),
        out_specs=pl.BlockSpec(memory_space=pltpu.SMEM),
    )(S, deg.reshape(NC, Npad, 1), b4r, features)

    return loss.reshape(())


# speedup vs baseline: 1.0579x; 1.0579x over previous
import functools

import jax
import jax.numpy as jnp
from jax import lax
from jax.experimental import pallas as pl
from jax.experimental.pallas import tpu as pltpu
from jax.experimental.pallas import tpu_sc as plsc

NC = 2
NS = 16
L = 16
NW = NC * NS

_f32 = jnp.float32
_i32 = jnp.int32


def _sc_mesh():
    return plsc.VectorSubcoreMesh(core_axis_name="c", subcore_axis_name="s")


_SC_PARAMS = pltpu.CompilerParams(needs_layout_passes=False,
                                  use_tc_tiling_on_sc=False)


@functools.lru_cache(maxsize=None)
def _make_w_kernel(Npad, ETp, C, CPT):
    per_tile = CPT * C

    assert CPT % 2 == 0

    def body(s1_hbm, d1_hbm, src_hbm, dst_hbm, w_hbm, s1_v, d1_v, src0,
             src1, dst0, dst1, w0, w1, isem, wsem0, wsem1):
        srcb = (src0, src1)
        dstb = (dst0, dst1)
        wb = (w0, w1)
        wsem = (wsem0, wsem1)
        cid = lax.axis_index("c")
        sid = lax.axis_index("s")
        base_r = (cid * NS + sid) * CPT
        pltpu.sync_copy(s1_hbm, s1_v)
        pltpu.sync_copy(d1_hbm, d1_v)

        def load_idx(j, b, sync):
            r = base_r + j
            cps = [pltpu.make_async_copy(src_hbm.at[r], srcb[b], isem),
                   pltpu.make_async_copy(dst_hbm.at[r], dstb[b], isem)]
            for cp in cps:
                cp.start()
            if sync:
                for cp in cps:
                    cp.wait()

        def wait_idx(b):
            pltpu.make_async_copy(src_hbm.at[0], srcb[b], isem).wait()
            pltpu.make_async_copy(dst_hbm.at[0], dstb[b], isem).wait()

        load_idx(0, 0, True)
        load_idx(1, 1, False)

        def pair(k, carry):
            for b in range(2):
                j = 2 * k + b

                @pl.when(k > 0)
                def _():
                    pltpu.make_async_copy(
                        wb[b], w_hbm.at[pl.ds(0, C)], wsem[b]).wait()

                for q in range(C // L):
                    si = srcb[b][pl.ds(q * L, L)]
                    di = dstb[b][pl.ds(q * L, L)]
                    sv = plsc.load_gather(s1_v, [si])
                    dv = plsc.load_gather(d1_v, [di])
                    e = 1.0 / (1.0 + jnp.exp(-(sv + dv)))
                    wb[b][pl.ds(q * L, L)] = jnp.exp(e)
                pltpu.async_copy(
                    wb[b], w_hbm.at[pl.ds((base_r + j) * C, C)], wsem[b])

                @pl.when(2 * k + 1 + b < CPT)
                def _():
                    wait_idx(1 - b)

                @pl.when(2 * k + 2 + b < CPT)
                def _():
                    load_idx(j + 2, b, False)
            return carry

        lax.fori_loop(0, CPT // 2, pair, 0)
        for b in range(2):
            pltpu.make_async_copy(wb[b], w_hbm.at[pl.ds(0, C)],
                                  wsem[b]).wait()

    return pl.kernel(
        body,
        out_type=jax.ShapeDtypeStruct((ETp,), _f32),
        mesh=_sc_mesh(),
        compiler_params=_SC_PARAMS,
        scratch_types=[
            pltpu.VMEM((Npad,), _f32),
            pltpu.VMEM((Npad,), _f32),
            pltpu.VMEM((C,), _i32),
            pltpu.VMEM((C,), _i32),
            pltpu.VMEM((C,), _i32),
            pltpu.VMEM((C,), _i32),
            pltpu.VMEM((C,), _f32),
            pltpu.VMEM((C,), _f32),
            pltpu.SemaphoreType.DMA,
            pltpu.SemaphoreType.DMA,
            pltpu.SemaphoreType.DMA,
        ],
    )


@functools.lru_cache(maxsize=None)
def _make_spmm_kernel(Npad, ETp, C, CPT, K, with_w, want_z=True,
                      compute_w=False):
    assert C == 128 and CPT % 6 == 0
    RPT = Npad // NS
    SB = 128
    NB = RPT // SB

    def body(h_hbm, src_hbm, dst_hbm, *rest):
        s1_hbm = d1_hbm = w_hbm = w_out = None
        if compute_w:
            s1_hbm, d1_hbm, rest = rest[0], rest[1], rest[2:]
        elif with_w:
            w_hbm, rest = rest[0], rest[1:]
        if want_z:
            acc_out, z_out, rest = rest[0], rest[1], rest[2:]
        else:
            acc_out, rest = rest[0], rest[1:]
            z_out = None
        if compute_w:
            w_out, rest = rest[0], rest[1:]
        if want_z:
            z_sh, zst_v, rest = rest[0], rest[1], rest[2:]
        else:
            z_sh = zst_v = None
        if compute_w:
            (s1c0, s1c1, d1c0, d1c1, esem0, esem1,
             rest) = (*rest[:6], rest[6:])
            s1c = (s1c0, s1c1)
            d1c = (d1c0, d1c1)
            esem = (esem0, esem1)
        (acc_sh, rows0, rows1,
         si0, si1, si2, di0, di1, di2, w0, w1, w2,
         gsem0, gsem1, ssem0, ssem1, isem) = rest
        h_src = h_hbm
        stage_v = rows0.at[pl.ds(0, SB)]
        rows = (rows0, rows1)
        si = (si0, si1, si2)
        di = (di0, di1, di2)
        wv = (w0, w1, w2)
        gsem = (gsem0, gsem1)
        ssem = (ssem0, ssem1)
        cid = lax.axis_index("c")
        sid = lax.axis_index("s")
        base_r = (cid * NS + sid) * CPT
        myrow = sid * RPT
        zero = jnp.zeros((L,), _f32)

        def zr(r, carry):
            for q in range(K // L):
                rows0[r, pl.ds(q * L, L)] = zero
            return carry

        lax.fori_loop(0, SB, zr, 0)
        for b in range(NB):
            pltpu.sync_copy(stage_v, acc_sh.at[pl.ds(myrow + b * SB, SB)])

        if want_z:
            def zz(r, carry):
                zst_v[pl.ds(r * L, L)] = zero
                return carry

            lax.fori_loop(0, RPT // L, zz, 0)
            pltpu.sync_copy(zst_v, z_sh.at[pl.ds(myrow, RPT)])

        if not with_w and want_z:
            one = jnp.full((L,), 1.0, _f32)

            def oo(r, carry):
                w0[pl.ds(r * L, L)] = one
                w1[pl.ds(r * L, L)] = one
                w2[pl.ds(r * L, L)] = one
                return carry

            lax.fori_loop(0, C // L, oo, 0)

        plsc.subcore_barrier()

        def load_idx(j, s3, sync):
            r = base_r + j
            cps = [pltpu.make_async_copy(src_hbm.at[r], si[s3], isem),
                   pltpu.make_async_copy(dst_hbm.at[r], di[s3], isem)]
            if with_w and not compute_w:
                cps.append(pltpu.make_async_copy(
                    w_hbm.at[pl.ds(r * C, C)], wv[s3], isem))
            for cp in cps:
                cp.start()
            if sync:
                for cp in cps:
                    cp.wait()

        def wait_idx(s3):
            pltpu.make_async_copy(src_hbm.at[0], si[s3], isem).wait()
            pltpu.make_async_copy(dst_hbm.at[0], di[s3], isem).wait()
            if with_w and not compute_w:
                pltpu.make_async_copy(w_hbm.at[pl.ds(0, C)], wv[s3],
                                      isem).wait()

        def start_egather(b2, s3):
            pltpu.async_copy(s1_hbm.at[si[s3]], s1c[b2], esem[b2])
            pltpu.async_copy(d1_hbm.at[di[s3]], d1c[b2], esem[b2])

        def wait_egather(b2, s3):
            pltpu.make_async_copy(s1_hbm.at[si[s3]], s1c[b2],
                                  esem[b2]).wait()
            pltpu.make_async_copy(d1_hbm.at[di[s3]], d1c[b2],
                                  esem[b2]).wait()

        def start_gather(b2, s3):
            pltpu.async_copy(h_src.at[si[s3]], rows[b2], gsem[b2])

        def wait_gather(b2, s3):
            pltpu.make_async_copy(h_src.at[si[s3]], rows[b2],
                                  gsem[b2]).wait()

        def start_scatter(b2, s3):
            pltpu.async_copy(rows[b2], acc_sh.at[di[s3]], ssem[b2],
                             add=True)
            if want_z:
                pltpu.async_copy(wv[s3], z_sh.at[di[s3]], ssem[b2],
                                 add=True)

        def wait_scatter(b2, s3):
            pltpu.make_async_copy(rows[b2], acc_sh.at[di[s3]],
                                  ssem[b2]).wait()
            if want_z:
                pltpu.make_async_copy(wv[s3], z_sh.at[di[s3]],
                                      ssem[b2]).wait()
            if compute_w:
                pltpu.make_async_copy(wv[s3], w_out.at[pl.ds(0, C)],
                                      ssem[b2]).wait()

        load_idx(0, 0, True)
        start_gather(0, 0)
        if compute_w:
            start_egather(0, 0)
        load_idx(1, 1, False)

        def six(k, carry):
            for u in range(6):
                j = 6 * k + u
                b2 = u % 2
                s3 = u % 3
                wait_gather(b2, s3)
                if u == 0:
                    @pl.when(k > 0)
                    def _():
                        wait_scatter(1 - b2, (u + 2) % 3)
                else:
                    wait_scatter(1 - b2, (u + 2) % 3)

                @pl.when(j + 1 < CPT)
                def _():
                    wait_idx((u + 1) % 3)
                    start_gather(1 - b2, (u + 1) % 3)
                    if compute_w:
                        start_egather(1 - b2, (u + 1) % 3)

                @pl.when(j + 2 < CPT)
                def _():
                    load_idx(j + 2, (u + 2) % 3, False)

                if compute_w:
                    wait_egather(b2, s3)
                    for q in range(C // L):
                        sv = s1c[b2][pl.ds(q * L, L)]
                        dv = d1c[b2][pl.ds(q * L, L)]
                        e = 1.0 / (1.0 + jnp.exp(-(sv + dv)))
                        wv[s3][pl.ds(q * L, L)] = jnp.exp(e)
                    pltpu.async_copy(
                        wv[s3], w_out.at[pl.ds((base_r + j) * C, C)],
                        ssem[b2])
                if with_w:
                    zvec = jnp.zeros((L,), _i32)

                    def sc_row(r, c2):
                        wspl = plsc.load_gather(wv[s3], [zvec + r])
                        for q in range(K // L):
                            rows[b2][r, pl.ds(q * L, L)] = (
                                rows[b2][r, pl.ds(q * L, L)] * wspl)
                        return c2

                    lax.fori_loop(0, C, sc_row, 0, unroll=4)
                start_scatter(b2, s3)
            return carry

        lax.fori_loop(0, CPT // 6, six, 0)
        wait_scatter((CPT - 1) % 2, (CPT - 1) % 3)
        plsc.subcore_barrier()

        pltpu.sync_copy(acc_sh.at[pl.ds(myrow, RPT)],
                        acc_out.at[cid, pl.ds(myrow, RPT)])
        if want_z:
            pltpu.sync_copy(z_sh.at[pl.ds(myrow, RPT)],
                            z_out.at[cid, pl.ds(myrow, RPT)])

    scratch = ([pltpu.VMEM_SHARED((Npad,), _f32),
                pltpu.VMEM((RPT,), _f32)] if want_z else []) + \
        ([pltpu.VMEM((C,), _f32), pltpu.VMEM((C,), _f32),
          pltpu.VMEM((C,), _f32), pltpu.VMEM((C,), _f32),
          pltpu.SemaphoreType.DMA, pltpu.SemaphoreType.DMA]
         if compute_w else []) + [
        pltpu.VMEM_SHARED((Npad, K), _f32),
        pltpu.VMEM((C, K), _f32),
        pltpu.VMEM((C, K), _f32),
        pltpu.VMEM((C,), _i32),
        pltpu.VMEM((C,), _i32),
        pltpu.VMEM((C,), _i32),
        pltpu.VMEM((C,), _i32),
        pltpu.VMEM((C,), _i32),
        pltpu.VMEM((C,), _i32),
        pltpu.VMEM((C,), _f32),
        pltpu.VMEM((C,), _f32),
        pltpu.VMEM((C,), _f32),
        pltpu.SemaphoreType.DMA,
        pltpu.SemaphoreType.DMA,
        pltpu.SemaphoreType.DMA,
        pltpu.SemaphoreType.DMA,
        pltpu.SemaphoreType.DMA,
    ]
    out_type = [jax.ShapeDtypeStruct((NC, Npad, K), _f32)]
    if want_z:
        out_type.append(jax.ShapeDtypeStruct((NC, Npad), _f32))
    if compute_w:
        out_type.append(jax.ShapeDtypeStruct((ETp,), _f32))
    out_type = tuple(out_type) if len(out_type) > 1 else out_type[0]
    return pl.kernel(
        body,
        out_type=out_type,
        mesh=_sc_mesh(),
        compiler_params=_SC_PARAMS,
        scratch_types=scratch,
    )


def _tc1(x_ref, ws_ref, wd_ref, as_ref, ad_ref, hs_ref, s1_ref, d1_ref):
    x = x_ref[...]
    hs = jnp.dot(x, ws_ref[...], preferred_element_type=_f32)
    hs_ref[...] = hs
    s1_ref[...] = jnp.dot(hs, as_ref[...], preferred_element_type=_f32)
    vd = jnp.dot(wd_ref[...], ad_ref[...], preferred_element_type=_f32)
    d1_ref[...] = jnp.dot(x, vd, preferred_element_type=_f32)


def _tc2(P_ref, z_ref, b1_ref, w2_ref, out_ref):
    Ps = P_ref[0] + P_ref[1]
    z = z_ref[0] + z_ref[1]
    h1 = Ps * (1.0 / z) + b1_ref[...]
    h1 = jnp.where(h1 > 0, h1, jnp.exp(jnp.minimum(h1, 0.0)) - 1.0)
    out_ref[...] = jnp.dot(h1, w2_ref[...], preferred_element_type=_f32)


def _tc3(Q_ref, zd_ref, b2_ref, w2t_ref, out_ref):
    agg = Q_ref[0] + Q_ref[1]
    deg = zd_ref[0] + zd_ref[1]
    h2 = agg * (1.0 / deg) + b2_ref[...]
    nrm = jnp.sqrt(jnp.sum(h2 * h2, axis=1, keepdims=True))
    h2 = h2 / jnp.maximum(nrm, 1e-12)
    out_ref[...] = jnp.dot(h2, w2t_ref[...], preferred_element_type=_f32)


def _tc4(R_ref, z_ref, b3_ref, w1t_ref, out_ref):
    Rs = R_ref[0] + R_ref[1]
    z = z_ref[0] + z_ref[1]
    h3 = Rs * (1.0 / z) + b3_ref[...]
    h3 = jnp.where(h3 > 0, h3, jnp.exp(jnp.minimum(h3, 0.0)) - 1.0)
    out_ref[...] = jnp.dot(h3, w1t_ref[...], preferred_element_type=_f32)


def _make_tc5(N, D):
    def _tc5(S_ref, zd_ref, b4_ref, x_ref, loss_ref):
        agg = S_ref[0, :N, :] + S_ref[1, :N, :]
        deg = zd_ref[0, :N, :] + zd_ref[1, :N, :]
        h4 = agg * (1.0 / deg) + b4_ref[...]
        r = x_ref[...] - h4
        loss_ref[0, 0] = jnp.sum(r * r) / (N * D)

    return _tc5


def kernel(features, edge_index, W1_src, W1_dst, a1_src, a1_dst, b1,
           W2_src, W2_dst, b2, b3, b4):
    N, D = features.shape
    F1 = W1_src.shape[1]
    F2 = W2_src.shape[1]
    E = edge_index.shape[1]
    Npad = -(-N // 2048) * 2048
    C = 128
    Etot = E + N
    CPT = -(-Etot // (NW * C))
    CPT = -(-CPT // 6) * 6
    ETp = NW * C * CPT
    NCH = ETp // 128
    npadE = ETp - Etot

    xp = jnp.zeros((Npad, D), _f32).at[:N].set(features)
    loop = jnp.arange(N, dtype=_i32)
    pad_src = jnp.arange(npadE, dtype=_i32) % N
    pad_dst = N + jnp.arange(npadE, dtype=_i32) % (Npad - N)
    src = jnp.concatenate([edge_index[0].astype(_i32), loop,
                           pad_src]).reshape(NCH, 128)
    dst = jnp.concatenate([edge_index[1].astype(_i32), loop,
                           pad_dst]).reshape(NCH, 128)
    a_s = a1_src.reshape(F1, 1)
    a_d = a1_dst.reshape(F1, 1)
    b1r = b1.reshape(1, F1)
    b2r = b2.reshape(1, F2)
    b3r = b3.reshape(1, F1)
    b4r = b4.reshape(1, D)
    W2T = W2_src.T
    W1T = W1_src.T

    hs1, s1, d1 = pl.pallas_call(
        _tc1,
        out_shape=(jax.ShapeDtypeStruct((Npad, F1), _f32),
                   jax.ShapeDtypeStruct((Npad, 1), _f32),
                   jax.ShapeDtypeStruct((Npad, 1), _f32)),
    )(xp, W1_src, W1_dst, a_s, a_d)

    P, z, w = _make_spmm_kernel(Npad, ETp, C, CPT, F1, True,
                                compute_w=True)(
        hs1, src, dst, s1.reshape(Npad), d1.reshape(Npad))

    h2pre = pl.pallas_call(
        _tc2, out_shape=jax.ShapeDtypeStruct((Npad, F2), _f32),
    )(P, z.reshape(NC, Npad, 1), b1r, W2_src)

    Q, deg = _make_spmm_kernel(Npad, ETp, C, CPT, F2, False)(h2pre, src, dst)

    h3pre = pl.pallas_call(
        _tc3, out_shape=jax.ShapeDtypeStruct((Npad, F1), _f32),
    )(Q, deg.reshape(NC, Npad, 1), b2r, W2T)

    R = _make_spmm_kernel(Npad, ETp, C, CPT, F1, True,
                          want_z=False)(h3pre, src, dst, w)

    h4pre = pl.pallas_call(
        _tc4, out_shape=jax.ShapeDtypeStruct((Npad, D), _f32),
    )(R, z.reshape(NC, Npad, 1), b3r, W1T)

    S = _make_spmm_kernel(Npad, ETp, C, CPT, D, False,
                          want_z=False)(h4pre, src, dst)

    loss = pl.pallas_call(
        _make_tc5(N, D),
        out_shape=jax.ShapeDtypeStruct((1, 1), _f32),
        out_specs=pl.BlockSpec(memory_space=pltpu.SMEM),
    )(S, deg.reshape(NC, Npad, 1), b4r, features)

    return loss.reshape(())
